# baseline (speedup 1.0000x reference)
import functools

import jax
import jax.numpy as jnp
from jax import lax
from jax.experimental import pallas as pl
from jax.experimental.pallas import tpu as pltpu
from jax.experimental.pallas import tpu_sc as plsc

L = 16
NG = 8
NEG_INF = float("-inf")


def _insert(state, v, vs=None):
    acc, m1, m2, m3 = state
    acc = acc + (v if vs is None else vs)
    hi1 = jnp.maximum(m1, v)
    lo1 = jnp.minimum(m1, v)
    hi2 = jnp.maximum(m2, lo1)
    lo2 = jnp.minimum(m2, lo1)
    hi3 = jnp.maximum(m3, lo2)
    return (acc, hi1, hi2, hi3)


def _merge_states(a, b):
    acc, m1, m2, m3 = a
    b_acc, b1, b2, b3 = b
    acc = acc + b_acc
    _, m1, m2, m3 = _insert((acc, m1, m2, m3), b1, vs=jnp.zeros_like(b1))
    hi2 = jnp.maximum(m2, b2)
    lo2 = jnp.minimum(m2, b2)
    m2, m3 = hi2, jnp.maximum(m3, lo2)
    m3 = jnp.maximum(m3, jnp.minimum(m2, b3))
    return acc, m1, m2, m3


def _chunk_reduce(buf, n_vec, states):
    def body(jj, sts):
        return tuple(
            _insert(sts[g], buf[jj, pl.ds(g * L, L)]) for g in range(NG)
        )

    return lax.fori_loop(0, n_vec, body, states, unroll=2)


def _make_sc_call(N, R):
    info = plsc.get_sparse_core_info()
    NC, NS = info.num_cores, info.num_subcores
    NW = NC * NS
    assert R == NG * L
    n_tiles = N // 8
    main_tiles = n_tiles // NW * NW
    MAIN = main_tiles * 8
    STRIPE = MAIN // NW
    tail_tiles = n_tiles - main_tiles
    assert tail_tiles <= NW and (N - MAIN) == tail_tiles * 8
    NCH = 10
    CJ = STRIPE // NCH
    assert CJ % 8 == 0 and CJ * NCH == STRIPE

    mesh = plsc.VectorSubcoreMesh(core_axis_name="c", subcore_axis_name="s")

    @functools.partial(
        pl.kernel,
        out_type=jax.ShapeDtypeStruct((NC, NG, 4, L), jnp.float32),
        mesh=mesh,
        compiler_params=pltpu.CompilerParams(needs_layout_passes=False),
        scratch_types=[
            pltpu.VMEM((CJ, R), jnp.float32),
            pltpu.VMEM((CJ, R), jnp.float32),
            pltpu.VMEM((8, R), jnp.float32),
            pltpu.VMEM((NG, 4, L), jnp.float32),
            pltpu.VMEM((NS, 4, L), jnp.float32),
            pltpu.VMEM((4, L), jnp.float32),
            pltpu.VMEM_SHARED((NG, NS, 4, L), jnp.float32),
            pltpu.SemaphoreType.DMA,
            pltpu.SemaphoreType.DMA,
            pltpu.SemaphoreType.DMA,
        ],
    )
    def sc_call(
        xt_hbm, out_hbm, buf0, buf1, tailbuf, statebuf, gatherbuf, mergebuf,
        shared, sem0, sem1, semt,
    ):
        c = lax.axis_index("c")
        s = lax.axis_index("s")
        w = c * NS + s
        j0 = w * STRIPE
        bufs = (buf0, buf1)
        sems = (sem0, sem1)

        def copy(k):
            return pltpu.make_async_copy(
                xt_hbm.at[pl.ds(j0 + k * CJ, CJ)], bufs[k % 2], sems[k % 2]
            )

        tw = jnp.where(w < tail_tiles, w, w - tail_tiles)
        tail_copy = pltpu.make_async_copy(
            xt_hbm.at[pl.ds(MAIN + 8 * tw, 8)], tailbuf, semt
        )
        copy(0).start()
        tail_copy.start()

        zeros = jnp.zeros((L,), jnp.float32)
        ninf = jnp.full((L,), NEG_INF)
        states = tuple((zeros, ninf, ninf, ninf) for _ in range(NG))

        for k in range(NCH):
            if k + 1 < NCH:
                copy(k + 1).start()
            copy(k).wait()
            states = _chunk_reduce(bufs[k % 2], CJ, states)

        tail_copy.wait()
        valid = w < tail_tiles
        states = list(states)
        for jj in range(8):
            for g in range(NG):
                v = tailbuf[jj, pl.ds(g * L, L)]
                vt = jnp.where(valid, v, NEG_INF)
                vs = jnp.where(valid, v, 0.0)
                states[g] = _insert(states[g], vt, vs=vs)

        for g in range(NG):
            acc, m1, m2, m3 = states[g]
            statebuf[g, 0] = acc
            statebuf[g, 1] = m1
            statebuf[g, 2] = m2
            statebuf[g, 3] = m3
            pltpu.sync_copy(statebuf.at[g], shared.at[g, s])

        plsc.subcore_barrier()

        @pl.when(s < NG)
        def _reduce():
            pltpu.sync_copy(shared.at[s], gatherbuf)
            acc = gatherbuf[0, 0]
            m1 = gatherbuf[0, 1]
            m2 = gatherbuf[0, 2]
            m3 = gatherbuf[0, 3]
            st = (acc, m1, m2, m3)
            for w2 in range(1, NS):
                other = (
                    gatherbuf[w2, 0], gatherbuf[w2, 1],
                    gatherbuf[w2, 2], gatherbuf[w2, 3],
                )
                st = _merge_states(st, other)
            mergebuf[0] = st[0]
            mergebuf[1] = st[1]
            mergebuf[2] = st[2]
            mergebuf[3] = st[3]
            pltpu.sync_copy(mergebuf, out_hbm.at[c, s])

    return sc_call


def kernel(x, head_len):
    del head_len
    R, N = x.shape
    out = _make_sc_call(N, R)(x.T)
    a = (out[0, :, 0], out[0, :, 1], out[0, :, 2], out[0, :, 3])
    b = (out[1, :, 0], out[1, :, 1], out[1, :, 2], out[1, :, 3])
    acc, _m1, m2, m3 = _merge_states(a, b)
    return jnp.sum(acc - m2 - 2.0 * m3)
